# baseline (speedup 1.0000x reference)
import functools
import jax
import jax.numpy as jnp
from jax import lax
from jax.experimental import pallas as pl
from jax.experimental.pallas import tpu as pltpu
from jax.experimental.pallas import tpu_sc as plsc

N = 10000
E = 320000
D = 64
NC = 2
NS = 16
NW = NC * NS
CH = 128
NCHW = 80
E_PAD = NW * NCHW * CH
ACC_N = 10112
ROWS_PER_SUB = ACC_N // NS
DEG_W = 16
BLK = 1000
GRID = N // BLK

_mesh = plsc.VectorSubcoreMesh(core_axis_name="c", subcore_axis_name="s")


def _sc_deg_body(dst_hbm, ones_hbm, zero_hbm, out_hbm, didx, ones_v, acc, sem):
    c = lax.axis_index("c")
    s = lax.axis_index("s")
    w = s * NC + c
    r0 = s * ROWS_PER_SUB
    pltpu.sync_copy(zero_hbm.at[pl.ds(r0, ROWS_PER_SUB)],
                    acc.at[pl.ds(r0, ROWS_PER_SUB)])
    pltpu.sync_copy(dst_hbm.at[pl.ds(w * NCHW, NCHW)], didx)
    pltpu.sync_copy(ones_hbm, ones_v)
    plsc.subcore_barrier()

    def body(j, carry):
        @pl.when(j >= 16)
        def _():
            pltpu.make_async_copy(ones_v, acc.at[didx.at[0]], sem).wait()

        pltpu.async_copy(ones_v, acc.at[didx.at[j]], sem, add=True)
        return carry

    lax.fori_loop(0, NCHW, body, 0)

    def drain(j, carry):
        pltpu.make_async_copy(ones_v, acc.at[didx.at[0]], sem).wait()
        return carry

    lax.fori_loop(0, 16, drain, 0)
    plsc.subcore_barrier()
    pltpu.sync_copy(acc.at[pl.ds(r0, ROWS_PER_SUB)],
                    out_hbm.at[c, pl.ds(r0, ROWS_PER_SUB)])


_sc_deg = pl.kernel(
    _sc_deg_body,
    out_type=jax.ShapeDtypeStruct((NC, ACC_N, DEG_W), jnp.float32),
    mesh=_mesh,
    scratch_types=[
        pltpu.VMEM((NCHW, CH), jnp.int32),
        pltpu.VMEM((CH, DEG_W), jnp.float32),
        pltpu.VMEM_SHARED((ACC_N, DEG_W), jnp.float32),
        pltpu.SemaphoreType.DMA,
    ],
)


NBUF = 3
LOOKAHEAD = 2


def _sc_conv_body(hs_hbm, src_hbm, dst_hbm, zero_hbm, out_hbm,
                  sidx, didx, rows, hs_sp, acc, gsem, ssem):
    c = lax.axis_index("c")
    s = lax.axis_index("s")
    w = s * NC + c
    r0 = s * ROWS_PER_SUB
    pltpu.sync_copy(zero_hbm.at[pl.ds(r0, ROWS_PER_SUB)],
                    acc.at[pl.ds(r0, ROWS_PER_SUB)])
    @pl.when(s < 10)
    def _():
        pltpu.sync_copy(hs_hbm.at[pl.ds(s * 1000, 1000)],
                        hs_sp.at[pl.ds(s * 1000, 1000)])

    pltpu.sync_copy(src_hbm.at[pl.ds(w * NCHW, NCHW)], sidx)
    pltpu.sync_copy(dst_hbm.at[pl.ds(w * NCHW, NCHW)], didx)
    plsc.subcore_barrier()

    def body(j, carry):
        b = lax.rem(j, NBUF)

        @pl.when(jnp.logical_and(j >= NBUF, j < NCHW))
        def _():
            pltpu.make_async_copy(rows.at[b], acc.at[didx.at[j]],
                                  ssem.at[b]).wait()

        @pl.when(j < NCHW)
        def _():
            pltpu.async_copy(hs_sp.at[sidx.at[j]], rows.at[b], gsem.at[b])

        jk = j - LOOKAHEAD

        @pl.when(jk >= 0)
        def _():
            bk = lax.rem(jk, NBUF)
            pltpu.make_async_copy(hs_sp.at[sidx.at[jk]], rows.at[bk],
                                  gsem.at[bk]).wait()
            pltpu.async_copy(rows.at[bk], acc.at[didx.at[jk]], ssem.at[bk],
                             add=True)

        return carry

    lax.fori_loop(0, NCHW + LOOKAHEAD, body, 0)

    def drain(b, carry):
        pltpu.make_async_copy(rows.at[b], acc.at[didx.at[0]],
                              ssem.at[b]).wait()
        return carry

    lax.fori_loop(NCHW % NBUF, NCHW % NBUF + NBUF, lambda b, c: drain(
        lax.rem(b, NBUF), c), 0)
    plsc.subcore_barrier()
    pltpu.sync_copy(acc.at[pl.ds(r0, ROWS_PER_SUB)],
                    out_hbm.at[c, pl.ds(r0, ROWS_PER_SUB)])


_sc_conv = pl.kernel(
    _sc_conv_body,
    out_type=jax.ShapeDtypeStruct((NC, ACC_N, D), jnp.float32),
    mesh=_mesh,
    compiler_params=pltpu.CompilerParams(use_tc_tiling_on_sc=False),
    scratch_types=[
        pltpu.VMEM((NCHW, CH), jnp.int32),
        pltpu.VMEM((NCHW, CH), jnp.int32),
        pltpu.VMEM((NBUF, CH, D), jnp.float32),
        pltpu.VMEM_SHARED((N, D), jnp.float32),
        pltpu.VMEM_SHARED((ACC_N, D), jnp.float32),
        pltpu.SemaphoreType.DMA((NBUF,)),
        pltpu.SemaphoreType.DMA((NBUF,)),
    ],
)


def _tc_a_body(x_ref, w0_ref, deg_ref, hs0_ref, dinv_ref):
    deg = deg_ref[0, :, 0:1] + deg_ref[1, :, 0:1] + 1.0
    dinv = lax.rsqrt(deg)
    h0 = jnp.dot(x_ref[...], w0_ref[...], preferred_element_type=jnp.float32)
    hs0_ref[...] = h0 * dinv
    dinv_ref[...] = dinv


def _tc_a(x, w0, degparts):
    return pl.pallas_call(
        _tc_a_body,
        grid=(GRID,),
        in_specs=[
            pl.BlockSpec((BLK, 128), lambda i: (i, 0)),
            pl.BlockSpec((128, D), lambda i: (0, 0)),
            pl.BlockSpec((NC, BLK, DEG_W), lambda i: (0, i, 0)),
        ],
        out_specs=[
            pl.BlockSpec((BLK, D), lambda i: (i, 0)),
            pl.BlockSpec((BLK, 1), lambda i: (i, 0)),
        ],
        out_shape=[
            jax.ShapeDtypeStruct((N, D), jnp.float32),
            jax.ShapeDtypeStruct((N, 1), jnp.float32),
        ],
    )(x, w0, degparts)


def _tc_b_body(acc_ref, hs_ref, dinv_ref, b_ref, mw0_ref, mb0_ref,
               mw1_ref, mb1_ref, wn_ref, hsn_ref, sum_ref):
    i = pl.program_id(0)
    dinv = dinv_ref[...]
    g = dinv * (acc_ref[0] + acc_ref[1] + hs_ref[...]) + b_ref[...]
    t = jnp.maximum(
        jnp.dot(g, mw0_ref[...], preferred_element_type=jnp.float32)
        + mb0_ref[...], 0.0)
    h = jnp.dot(t, mw1_ref[...], preferred_element_type=jnp.float32) + mb1_ref[...]
    hsn_ref[...] = jnp.dot(h, wn_ref[...], preferred_element_type=jnp.float32) * dinv

    @pl.when(i == 0)
    def _():
        sum_ref[...] = jnp.zeros_like(sum_ref)

    sum_ref[...] += jnp.sum(h, axis=0, keepdims=True)


def _tc_b(accparts, hs, dinv, b, mw0, mb0, mw1, mb1, wn):
    return pl.pallas_call(
        _tc_b_body,
        grid=(GRID,),
        in_specs=[
            pl.BlockSpec((NC, BLK, D), lambda i: (0, i, 0)),
            pl.BlockSpec((BLK, D), lambda i: (i, 0)),
            pl.BlockSpec((BLK, 1), lambda i: (i, 0)),
            pl.BlockSpec((1, D), lambda i: (0, 0)),
            pl.BlockSpec((D, 8), lambda i: (0, 0)),
            pl.BlockSpec((1, 8), lambda i: (0, 0)),
            pl.BlockSpec((8, D), lambda i: (0, 0)),
            pl.BlockSpec((1, D), lambda i: (0, 0)),
            pl.BlockSpec((D, D), lambda i: (0, 0)),
        ],
        out_specs=[
            pl.BlockSpec((BLK, D), lambda i: (i, 0)),
            pl.BlockSpec((1, D), lambda i: (0, 0)),
        ],
        out_shape=[
            jax.ShapeDtypeStruct((N, D), jnp.float32),
            jax.ShapeDtypeStruct((1, D), jnp.float32),
        ],
    )(accparts, hs, dinv, b, mw0, mb0, mw1, mb1, wn)


def _tc_c_body(acc_ref, hs_ref, dinv_ref, b_ref, mw0_ref, mb0_ref,
               mw1_ref, mb1_ref, s1_ref, w1t_ref, w2t_ref, bt_ref,
               out_ref, sum_ref):
    i = pl.program_id(0)
    dinv = dinv_ref[...]
    g = dinv * (acc_ref[0] + acc_ref[1] + hs_ref[...]) + b_ref[...]
    t = jnp.maximum(
        jnp.dot(g, mw0_ref[...], preferred_element_type=jnp.float32)
        + mb0_ref[...], 0.0)
    h = jnp.dot(t, mw1_ref[...], preferred_element_type=jnp.float32) + mb1_ref[...]

    @pl.when(i == 0)
    def _():
        sum_ref[...] = jnp.zeros_like(sum_ref)

    sum_ref[...] += jnp.sum(h, axis=0, keepdims=True)

    @pl.when(i == GRID - 1)
    def _():
        logits = (
            jnp.dot(s1_ref[...], w1t_ref[...], preferred_element_type=jnp.float32)
            + jnp.dot(sum_ref[...], w2t_ref[...], preferred_element_type=jnp.float32)
            + bt_ref[...])
        out_ref[...] = jax.nn.sigmoid(logits)


def _tc_c(accparts, hs, dinv, b, mw0, mb0, mw1, mb1, s1, w1t, w2t, bt):
    return pl.pallas_call(
        _tc_c_body,
        grid=(GRID,),
        in_specs=[
            pl.BlockSpec((NC, BLK, D), lambda i: (0, i, 0)),
            pl.BlockSpec((BLK, D), lambda i: (i, 0)),
            pl.BlockSpec((BLK, 1), lambda i: (i, 0)),
            pl.BlockSpec((1, D), lambda i: (0, 0)),
            pl.BlockSpec((D, 8), lambda i: (0, 0)),
            pl.BlockSpec((1, 8), lambda i: (0, 0)),
            pl.BlockSpec((8, D), lambda i: (0, 0)),
            pl.BlockSpec((1, D), lambda i: (0, 0)),
            pl.BlockSpec((1, D), lambda i: (0, 0)),
            pl.BlockSpec((D, 2), lambda i: (0, 0)),
            pl.BlockSpec((D, 2), lambda i: (0, 0)),
            pl.BlockSpec((1, 2), lambda i: (0, 0)),
        ],
        out_specs=[
            pl.BlockSpec((1, 2), lambda i: (0, 0)),
            pl.BlockSpec((1, D), lambda i: (0, 0)),
        ],
        out_shape=[
            jax.ShapeDtypeStruct((1, 2), jnp.float32),
            jax.ShapeDtypeStruct((1, D), jnp.float32),
        ],
    )(accparts, hs, dinv, b, mw0, mb0, mw1, mb1, s1, w1t, w2t, bt)


def kernel(x, edge_index, gcn0_W, gcn0_b, gcn1_W, gcn1_b,
           mlp0_W0, mlp0_b0, mlp0_W1, mlp0_b1,
           mlp1_W0, mlp1_b0, mlp1_W1, mlp1_b1,
           tcl_f0, tcl_f1, tcl_f2, tcl_b, pi_hidden,
           attend_W, attend_b, out_W, out_b):
    f32 = jnp.float32
    src = edge_index[0]
    dst = edge_index[1]
    pad = E_PAD - E
    src2d = jnp.concatenate([src, jnp.zeros((pad,), jnp.int32)]).reshape(
        NW * NCHW, CH)
    dst2d = jnp.concatenate([dst, jnp.full((pad,), N, jnp.int32)]).reshape(
        NW * NCHW, CH)

    ones_deg = jnp.ones((CH, DEG_W), f32)
    zero_deg = jnp.zeros((ACC_N, DEG_W), f32)
    zero_acc = jnp.zeros((ACC_N, D), f32)

    wA = attend_W[:8, 0]
    wB = attend_W[8:, 0]
    g0v = tcl_f0.T @ wA
    Cmat = (jnp.einsum('d,dyz->yz', wA, tcl_b)
            + jnp.einsum('f,fyz->yz', wB, pi_hidden) + attend_b[0])
    Cvec = Cmat.T.reshape(1, 64)
    Kmat = jnp.einsum('yb,zc->bczy', tcl_f1, tcl_f2).reshape(64, 64)
    Wtail = Kmat @ out_W
    bt = Cvec @ out_W + out_b[None, :]
    w1t = (g0v[0] / N) * Wtail
    w2t = (g0v[1] / N) * Wtail

    degparts = _sc_deg(dst2d, ones_deg, zero_deg)
    hs0, dinv = _tc_a(x, gcn0_W, degparts)
    acc0 = _sc_conv(hs0, src2d, dst2d, zero_acc)
    hs1, s1 = _tc_b(acc0, hs0, dinv, gcn0_b[None, :],
                    mlp0_W0, mlp0_b0[None, :], mlp0_W1, mlp0_b1[None, :],
                    gcn1_W)
    acc1 = _sc_conv(hs1, src2d, dst2d, zero_acc)
    out, _ = _tc_c(acc1, hs1, dinv, gcn1_b[None, :],
                   mlp1_W0, mlp1_b0[None, :], mlp1_W1, mlp1_b1[None, :],
                   s1, w1t, w2t, bt)
    return out


# speedup vs baseline: 1.0403x; 1.0403x over previous
import functools
import jax
import jax.numpy as jnp
from jax import lax
from jax.experimental import pallas as pl
from jax.experimental.pallas import tpu as pltpu
from jax.experimental.pallas import tpu_sc as plsc

N = 10000
E = 320000
D = 64
NC = 2
NS = 16
NW = NC * NS
CH = 128
EROWS = E // CH
CHB = EROWS // NW
CHX = EROWS - CHB * NW
IDXR = CHB + 1
ACC_N = 10112
ROWS_PER_SUB = ACC_N // NS
DEG_W = 16
BLK = 1000
GRID = N // BLK

_mesh = plsc.VectorSubcoreMesh(core_axis_name="c", subcore_axis_name="s")


def _sc_deg_body(dst_hbm, ones_hbm, zero_hbm, out_hbm, didx, ones_v, acc, sem):
    c = lax.axis_index("c")
    s = lax.axis_index("s")
    w = s * NC + c
    base = CHB * w + jnp.minimum(w, CHX)
    nch = CHB + (w < CHX).astype(jnp.int32)
    r0 = s * ROWS_PER_SUB
    pltpu.sync_copy(zero_hbm.at[pl.ds(r0, ROWS_PER_SUB)],
                    acc.at[pl.ds(r0, ROWS_PER_SUB)])
    pltpu.sync_copy(dst_hbm.at[pl.ds(base, CHB)], didx.at[pl.ds(0, CHB)])

    @pl.when(w < CHX)
    def _():
        pltpu.sync_copy(dst_hbm.at[pl.ds(base + CHB, 1)],
                        didx.at[pl.ds(CHB, 1)])

    pltpu.sync_copy(ones_hbm, ones_v)
    plsc.subcore_barrier()

    def body(j, carry):
        @pl.when(j >= 16)
        def _():
            pltpu.make_async_copy(ones_v, acc.at[didx.at[0]], sem).wait()

        pltpu.async_copy(ones_v, acc.at[didx.at[j]], sem, add=True)
        return carry

    lax.fori_loop(0, nch, body, 0)

    def drain(j, carry):
        pltpu.make_async_copy(ones_v, acc.at[didx.at[0]], sem).wait()
        return carry

    lax.fori_loop(0, 16, drain, 0)
    plsc.subcore_barrier()
    pltpu.sync_copy(acc.at[pl.ds(r0, ROWS_PER_SUB)],
                    out_hbm.at[c, pl.ds(r0, ROWS_PER_SUB)])


_sc_deg = pl.kernel(
    _sc_deg_body,
    out_type=jax.ShapeDtypeStruct((NC, ACC_N, DEG_W), jnp.float32),
    mesh=_mesh,
    compiler_params=pltpu.CompilerParams(use_tc_tiling_on_sc=False),
    scratch_types=[
        pltpu.VMEM((IDXR, CH), jnp.int32),
        pltpu.VMEM((CH, DEG_W), jnp.float32),
        pltpu.VMEM_SHARED((ACC_N, DEG_W), jnp.float32),
        pltpu.SemaphoreType.DMA,
    ],
)


NBUF = 3
LOOKAHEAD = 2


def _sc_conv_body(hs_hbm, src_hbm, dst_hbm, zero_hbm, out_hbm,
                  sidx, didx, rows, hs_sp, acc, gsem, ssem):
    c = lax.axis_index("c")
    s = lax.axis_index("s")
    w = s * NC + c
    base = CHB * w + jnp.minimum(w, CHX)
    nch = CHB + (w < CHX).astype(jnp.int32)
    r0 = s * ROWS_PER_SUB
    pltpu.sync_copy(zero_hbm.at[pl.ds(r0, ROWS_PER_SUB)],
                    acc.at[pl.ds(r0, ROWS_PER_SUB)])
    @pl.when(s < 10)
    def _():
        pltpu.sync_copy(hs_hbm.at[pl.ds(s * 1000, 1000)],
                        hs_sp.at[pl.ds(s * 1000, 1000)])

    pltpu.sync_copy(src_hbm.at[pl.ds(base, CHB)], sidx.at[pl.ds(0, CHB)])
    pltpu.sync_copy(dst_hbm.at[pl.ds(base, CHB)], didx.at[pl.ds(0, CHB)])

    @pl.when(w < CHX)
    def _():
        pltpu.sync_copy(src_hbm.at[pl.ds(base + CHB, 1)],
                        sidx.at[pl.ds(CHB, 1)])
        pltpu.sync_copy(dst_hbm.at[pl.ds(base + CHB, 1)],
                        didx.at[pl.ds(CHB, 1)])

    plsc.subcore_barrier()

    def body(j, carry):
        b = lax.rem(j, NBUF)

        @pl.when(jnp.logical_and(j >= NBUF, j < nch))
        def _():
            pltpu.make_async_copy(rows.at[b], acc.at[didx.at[j]],
                                  ssem.at[b]).wait()

        @pl.when(j < nch)
        def _():
            pltpu.async_copy(hs_sp.at[sidx.at[j]], rows.at[b], gsem.at[b])

        jk = j - LOOKAHEAD

        @pl.when(jk >= 0)
        def _():
            bk = lax.rem(jk, NBUF)
            pltpu.make_async_copy(hs_sp.at[sidx.at[jk]], rows.at[bk],
                                  gsem.at[bk]).wait()
            pltpu.async_copy(rows.at[bk], acc.at[didx.at[jk]], ssem.at[bk],
                             add=True)

        return carry

    lax.fori_loop(0, nch + LOOKAHEAD, body, 0)

    def drain(b, carry):
        pltpu.make_async_copy(rows.at[b], acc.at[didx.at[0]],
                              ssem.at[b]).wait()
        return carry

    lax.fori_loop(0, NBUF, drain, 0)
    plsc.subcore_barrier()
    pltpu.sync_copy(acc.at[pl.ds(r0, ROWS_PER_SUB)],
                    out_hbm.at[c, pl.ds(r0, ROWS_PER_SUB)])


_sc_conv = pl.kernel(
    _sc_conv_body,
    out_type=jax.ShapeDtypeStruct((NC, ACC_N, D), jnp.float32),
    mesh=_mesh,
    compiler_params=pltpu.CompilerParams(use_tc_tiling_on_sc=False),
    scratch_types=[
        pltpu.VMEM((IDXR, CH), jnp.int32),
        pltpu.VMEM((IDXR, CH), jnp.int32),
        pltpu.VMEM((NBUF, CH, D), jnp.float32),
        pltpu.VMEM_SHARED((N, D), jnp.float32),
        pltpu.VMEM_SHARED((ACC_N, D), jnp.float32),
        pltpu.SemaphoreType.DMA((NBUF,)),
        pltpu.SemaphoreType.DMA((NBUF,)),
    ],
)


def _tc_a_body(x_ref, w0_ref, deg_ref, hs0_ref, dinv_ref):
    deg = deg_ref[0, :, 0:1] + deg_ref[1, :, 0:1] + 1.0
    dinv = lax.rsqrt(deg)
    h0 = jnp.dot(x_ref[...], w0_ref[...], preferred_element_type=jnp.float32)
    hs0_ref[...] = h0 * dinv
    dinv_ref[...] = dinv


def _tc_a(x, w0, degparts):
    return pl.pallas_call(
        _tc_a_body,
        grid=(GRID,),
        in_specs=[
            pl.BlockSpec((BLK, 128), lambda i: (i, 0)),
            pl.BlockSpec((128, D), lambda i: (0, 0)),
            pl.BlockSpec((NC, BLK, DEG_W), lambda i: (0, i, 0)),
        ],
        out_specs=[
            pl.BlockSpec((BLK, D), lambda i: (i, 0)),
            pl.BlockSpec((BLK, 1), lambda i: (i, 0)),
        ],
        out_shape=[
            jax.ShapeDtypeStruct((N, D), jnp.float32),
            jax.ShapeDtypeStruct((N, 1), jnp.float32),
        ],
    )(x, w0, degparts)


def _tc_b_body(acc_ref, hs_ref, dinv_ref, b_ref, mw0_ref, mb0_ref,
               mw1_ref, mb1_ref, wn_ref, hsn_ref, sum_ref):
    i = pl.program_id(0)
    dinv = dinv_ref[...]
    g = dinv * (acc_ref[0] + acc_ref[1] + hs_ref[...]) + b_ref[...]
    t = jnp.maximum(
        jnp.dot(g, mw0_ref[...], preferred_element_type=jnp.float32)
        + mb0_ref[...], 0.0)
    h = jnp.dot(t, mw1_ref[...], preferred_element_type=jnp.float32) + mb1_ref[...]
    hsn_ref[...] = jnp.dot(h, wn_ref[...], preferred_element_type=jnp.float32) * dinv

    @pl.when(i == 0)
    def _():
        sum_ref[...] = jnp.zeros_like(sum_ref)

    sum_ref[...] += jnp.sum(h, axis=0, keepdims=True)


def _tc_b(accparts, hs, dinv, b, mw0, mb0, mw1, mb1, wn):
    return pl.pallas_call(
        _tc_b_body,
        grid=(GRID,),
        in_specs=[
            pl.BlockSpec((NC, BLK, D), lambda i: (0, i, 0)),
            pl.BlockSpec((BLK, D), lambda i: (i, 0)),
            pl.BlockSpec((BLK, 1), lambda i: (i, 0)),
            pl.BlockSpec((1, D), lambda i: (0, 0)),
            pl.BlockSpec((D, 8), lambda i: (0, 0)),
            pl.BlockSpec((1, 8), lambda i: (0, 0)),
            pl.BlockSpec((8, D), lambda i: (0, 0)),
            pl.BlockSpec((1, D), lambda i: (0, 0)),
            pl.BlockSpec((D, D), lambda i: (0, 0)),
        ],
        out_specs=[
            pl.BlockSpec((BLK, D), lambda i: (i, 0)),
            pl.BlockSpec((1, D), lambda i: (0, 0)),
        ],
        out_shape=[
            jax.ShapeDtypeStruct((N, D), jnp.float32),
            jax.ShapeDtypeStruct((1, D), jnp.float32),
        ],
    )(accparts, hs, dinv, b, mw0, mb0, mw1, mb1, wn)


def _tc_c_body(acc_ref, hs_ref, dinv_ref, b_ref, mw0_ref, mb0_ref,
               mw1_ref, mb1_ref, s1_ref, w1t_ref, w2t_ref, bt_ref,
               out_ref, sum_ref):
    i = pl.program_id(0)
    dinv = dinv_ref[...]
    g = dinv * (acc_ref[0] + acc_ref[1] + hs_ref[...]) + b_ref[...]
    t = jnp.maximum(
        jnp.dot(g, mw0_ref[...], preferred_element_type=jnp.float32)
        + mb0_ref[...], 0.0)
    h = jnp.dot(t, mw1_ref[...], preferred_element_type=jnp.float32) + mb1_ref[...]

    @pl.when(i == 0)
    def _():
        sum_ref[...] = jnp.zeros_like(sum_ref)

    sum_ref[...] += jnp.sum(h, axis=0, keepdims=True)

    @pl.when(i == GRID - 1)
    def _():
        logits = (
            jnp.dot(s1_ref[...], w1t_ref[...], preferred_element_type=jnp.float32)
            + jnp.dot(sum_ref[...], w2t_ref[...], preferred_element_type=jnp.float32)
            + bt_ref[...])
        out_ref[...] = jax.nn.sigmoid(logits)


def _tc_c(accparts, hs, dinv, b, mw0, mb0, mw1, mb1, s1, w1t, w2t, bt):
    return pl.pallas_call(
        _tc_c_body,
        grid=(GRID,),
        in_specs=[
            pl.BlockSpec((NC, BLK, D), lambda i: (0, i, 0)),
            pl.BlockSpec((BLK, D), lambda i: (i, 0)),
            pl.BlockSpec((BLK, 1), lambda i: (i, 0)),
            pl.BlockSpec((1, D), lambda i: (0, 0)),
            pl.BlockSpec((D, 8), lambda i: (0, 0)),
            pl.BlockSpec((1, 8), lambda i: (0, 0)),
            pl.BlockSpec((8, D), lambda i: (0, 0)),
            pl.BlockSpec((1, D), lambda i: (0, 0)),
            pl.BlockSpec((1, D), lambda i: (0, 0)),
            pl.BlockSpec((D, 2), lambda i: (0, 0)),
            pl.BlockSpec((D, 2), lambda i: (0, 0)),
            pl.BlockSpec((1, 2), lambda i: (0, 0)),
        ],
        out_specs=[
            pl.BlockSpec((1, 2), lambda i: (0, 0)),
            pl.BlockSpec((1, D), lambda i: (0, 0)),
        ],
        out_shape=[
            jax.ShapeDtypeStruct((1, 2), jnp.float32),
            jax.ShapeDtypeStruct((1, D), jnp.float32),
        ],
    )(accparts, hs, dinv, b, mw0, mb0, mw1, mb1, s1, w1t, w2t, bt)


def kernel(x, edge_index, gcn0_W, gcn0_b, gcn1_W, gcn1_b,
           mlp0_W0, mlp0_b0, mlp0_W1, mlp0_b1,
           mlp1_W0, mlp1_b0, mlp1_W1, mlp1_b1,
           tcl_f0, tcl_f1, tcl_f2, tcl_b, pi_hidden,
           attend_W, attend_b, out_W, out_b):
    f32 = jnp.float32
    src2d = edge_index[0].reshape(EROWS, CH)
    dst2d = edge_index[1].reshape(EROWS, CH)

    ones_deg = jnp.ones((CH, DEG_W), f32)
    zero_deg = jnp.zeros((ACC_N, DEG_W), f32)
    zero_acc = jnp.zeros((ACC_N, D), f32)

    wA = attend_W[:8, 0]
    wB = attend_W[8:, 0]
    g0v = tcl_f0.T @ wA
    Cmat = (jnp.einsum('d,dyz->yz', wA, tcl_b)
            + jnp.einsum('f,fyz->yz', wB, pi_hidden) + attend_b[0])
    Cvec = Cmat.T.reshape(1, 64)
    Kmat = jnp.einsum('yb,zc->bczy', tcl_f1, tcl_f2).reshape(64, 64)
    Wtail = Kmat @ out_W
    bt = Cvec @ out_W + out_b[None, :]
    w1t = (g0v[0] / N) * Wtail
    w2t = (g0v[1] / N) * Wtail

    degparts = _sc_deg(dst2d, ones_deg, zero_deg)
    hs0, dinv = _tc_a(x, gcn0_W, degparts)
    acc0 = _sc_conv(hs0, src2d, dst2d, zero_acc)
    hs1, s1 = _tc_b(acc0, hs0, dinv, gcn0_b[None, :],
                    mlp0_W0, mlp0_b0[None, :], mlp0_W1, mlp0_b1[None, :],
                    gcn1_W)
    acc1 = _sc_conv(hs1, src2d, dst2d, zero_acc)
    out, _ = _tc_c(acc1, hs1, dinv, gcn1_b[None, :],
                   mlp1_W0, mlp1_b0[None, :], mlp1_W1, mlp1_b1[None, :],
                   s1, w1t, w2t, bt)
    return out


# speedup vs baseline: 1.0632x; 1.0220x over previous
import functools
import jax
import jax.numpy as jnp
from jax import lax
from jax.experimental import pallas as pl
from jax.experimental.pallas import tpu as pltpu
from jax.experimental.pallas import tpu_sc as plsc

N = 10000
E = 320000
D = 64
NC = 2
NS = 16
NW = NC * NS
CH = 128
EROWS = E // CH
CHB = EROWS // NW
CHX = EROWS - CHB * NW
IDXR = CHB + 1
ACC_N = 10112
ROWS_PER_SUB = ACC_N // NS
DEG_W = 16
BLK = 1000
GRID = N // BLK

_mesh = plsc.VectorSubcoreMesh(core_axis_name="c", subcore_axis_name="s")


def _sc_deg_body(dst_hbm, ones_hbm, zero_hbm, out_hbm, didx, ones_v, acc, sem):
    c = lax.axis_index("c")
    s = lax.axis_index("s")
    w = s * NC + c
    base = CHB * w + jnp.minimum(w, CHX)
    nch = CHB + (w < CHX).astype(jnp.int32)
    r0 = s * ROWS_PER_SUB
    pltpu.async_copy(zero_hbm.at[pl.ds(r0, ROWS_PER_SUB)],
                     acc.at[pl.ds(r0, ROWS_PER_SUB)], sem)
    pltpu.async_copy(dst_hbm.at[pl.ds(base, CHB)], didx.at[pl.ds(0, CHB)],
                     sem)

    @pl.when(w < CHX)
    def _():
        pltpu.sync_copy(dst_hbm.at[pl.ds(base + CHB, 1)],
                        didx.at[pl.ds(CHB, 1)])

    pltpu.async_copy(ones_hbm, ones_v, sem)
    pltpu.make_async_copy(zero_hbm.at[pl.ds(r0, ROWS_PER_SUB)],
                          acc.at[pl.ds(r0, ROWS_PER_SUB)], sem).wait()
    pltpu.make_async_copy(dst_hbm.at[pl.ds(base, CHB)],
                          didx.at[pl.ds(0, CHB)], sem).wait()
    pltpu.make_async_copy(ones_hbm, ones_v, sem).wait()
    plsc.subcore_barrier()

    def body(j, carry):
        @pl.when(j >= 16)
        def _():
            pltpu.make_async_copy(ones_v, acc.at[didx.at[0]], sem).wait()

        pltpu.async_copy(ones_v, acc.at[didx.at[j]], sem, add=True)
        return carry

    lax.fori_loop(0, nch, body, 0)

    def drain(j, carry):
        pltpu.make_async_copy(ones_v, acc.at[didx.at[0]], sem).wait()
        return carry

    lax.fori_loop(0, 16, drain, 0)
    plsc.subcore_barrier()
    pltpu.sync_copy(acc.at[pl.ds(r0, ROWS_PER_SUB)],
                    out_hbm.at[c, pl.ds(r0, ROWS_PER_SUB)])


_sc_deg = pl.kernel(
    _sc_deg_body,
    out_type=jax.ShapeDtypeStruct((NC, ACC_N, DEG_W), jnp.float32),
    mesh=_mesh,
    compiler_params=pltpu.CompilerParams(use_tc_tiling_on_sc=False),
    scratch_types=[
        pltpu.VMEM((IDXR, CH), jnp.int32),
        pltpu.VMEM((CH, DEG_W), jnp.float32),
        pltpu.VMEM_SHARED((ACC_N, DEG_W), jnp.float32),
        pltpu.SemaphoreType.DMA,
    ],
)


NBUF = 3
LOOKAHEAD = 2


def _sc_conv_body(hs_hbm, src_hbm, dst_hbm, zero_hbm, out_hbm,
                  sidx, didx, rows, hs_sp, acc, gsem, ssem):
    c = lax.axis_index("c")
    s = lax.axis_index("s")
    w = s * NC + c
    base = CHB * w + jnp.minimum(w, CHX)
    nch = CHB + (w < CHX).astype(jnp.int32)
    r0 = s * ROWS_PER_SUB
    pltpu.async_copy(zero_hbm.at[pl.ds(r0, ROWS_PER_SUB)],
                     acc.at[pl.ds(r0, ROWS_PER_SUB)], gsem.at[0])

    @pl.when(s < 10)
    def _():
        pltpu.async_copy(hs_hbm.at[pl.ds(s * 1000, 1000)],
                         hs_sp.at[pl.ds(s * 1000, 1000)], gsem.at[1])

    pltpu.async_copy(src_hbm.at[pl.ds(base, CHB)], sidx.at[pl.ds(0, CHB)],
                     ssem.at[0])
    pltpu.async_copy(dst_hbm.at[pl.ds(base, CHB)], didx.at[pl.ds(0, CHB)],
                     ssem.at[1])

    @pl.when(w < CHX)
    def _():
        pltpu.sync_copy(src_hbm.at[pl.ds(base + CHB, 1)],
                        sidx.at[pl.ds(CHB, 1)])
        pltpu.sync_copy(dst_hbm.at[pl.ds(base + CHB, 1)],
                        didx.at[pl.ds(CHB, 1)])

    pltpu.make_async_copy(zero_hbm.at[pl.ds(r0, ROWS_PER_SUB)],
                          acc.at[pl.ds(r0, ROWS_PER_SUB)], gsem.at[0]).wait()

    @pl.when(s < 10)
    def _():
        pltpu.make_async_copy(hs_hbm.at[pl.ds(s * 1000, 1000)],
                              hs_sp.at[pl.ds(s * 1000, 1000)],
                              gsem.at[1]).wait()

    pltpu.make_async_copy(src_hbm.at[pl.ds(base, CHB)],
                          sidx.at[pl.ds(0, CHB)], ssem.at[0]).wait()
    pltpu.make_async_copy(dst_hbm.at[pl.ds(base, CHB)],
                          didx.at[pl.ds(0, CHB)], ssem.at[1]).wait()
    plsc.subcore_barrier()

    def body(j, carry):
        b = lax.rem(j, NBUF)

        @pl.when(jnp.logical_and(j >= NBUF, j < nch))
        def _():
            pltpu.make_async_copy(rows.at[b], acc.at[didx.at[j]],
                                  ssem.at[b]).wait()

        @pl.when(j < nch)
        def _():
            pltpu.async_copy(hs_sp.at[sidx.at[j]], rows.at[b], gsem.at[b])

        jk = j - LOOKAHEAD

        @pl.when(jk >= 0)
        def _():
            bk = lax.rem(jk, NBUF)
            pltpu.make_async_copy(hs_sp.at[sidx.at[jk]], rows.at[bk],
                                  gsem.at[bk]).wait()
            pltpu.async_copy(rows.at[bk], acc.at[didx.at[jk]], ssem.at[bk],
                             add=True)

        return carry

    lax.fori_loop(0, nch + LOOKAHEAD, body, 0)

    def drain(b, carry):
        pltpu.make_async_copy(rows.at[b], acc.at[didx.at[0]],
                              ssem.at[b]).wait()
        return carry

    lax.fori_loop(0, NBUF, drain, 0)
    plsc.subcore_barrier()
    pltpu.sync_copy(acc.at[pl.ds(r0, ROWS_PER_SUB)],
                    out_hbm.at[c, pl.ds(r0, ROWS_PER_SUB)])


_sc_conv = pl.kernel(
    _sc_conv_body,
    out_type=jax.ShapeDtypeStruct((NC, ACC_N, D), jnp.float32),
    mesh=_mesh,
    compiler_params=pltpu.CompilerParams(use_tc_tiling_on_sc=False),
    scratch_types=[
        pltpu.VMEM((IDXR, CH), jnp.int32),
        pltpu.VMEM((IDXR, CH), jnp.int32),
        pltpu.VMEM((NBUF, CH, D), jnp.float32),
        pltpu.VMEM_SHARED((N, D), jnp.float32),
        pltpu.VMEM_SHARED((ACC_N, D), jnp.float32),
        pltpu.SemaphoreType.DMA((NBUF,)),
        pltpu.SemaphoreType.DMA((NBUF,)),
    ],
)


def _tc_a_body(x_ref, w0_ref, deg_ref, hs0_ref, dinv_ref):
    deg = deg_ref[0, :, 0:1] + deg_ref[1, :, 0:1] + 1.0
    dinv = lax.rsqrt(deg)
    h0 = jnp.dot(x_ref[...], w0_ref[...], preferred_element_type=jnp.float32)
    hs0_ref[...] = h0 * dinv
    dinv_ref[...] = dinv


def _tc_a(x, w0, degparts):
    return pl.pallas_call(
        _tc_a_body,
        grid=(GRID,),
        in_specs=[
            pl.BlockSpec((BLK, 128), lambda i: (i, 0)),
            pl.BlockSpec((128, D), lambda i: (0, 0)),
            pl.BlockSpec((NC, BLK, DEG_W), lambda i: (0, i, 0)),
        ],
        out_specs=[
            pl.BlockSpec((BLK, D), lambda i: (i, 0)),
            pl.BlockSpec((BLK, 1), lambda i: (i, 0)),
        ],
        out_shape=[
            jax.ShapeDtypeStruct((N, D), jnp.float32),
            jax.ShapeDtypeStruct((N, 1), jnp.float32),
        ],
    )(x, w0, degparts)


def _tc_b_body(acc_ref, hs_ref, dinv_ref, b_ref, mw0_ref, mb0_ref,
               mw1_ref, mb1_ref, wn_ref, hsn_ref, sum_ref):
    i = pl.program_id(0)
    dinv = dinv_ref[...]
    g = dinv * (acc_ref[0] + acc_ref[1] + hs_ref[...]) + b_ref[...]
    t = jnp.maximum(
        jnp.dot(g, mw0_ref[...], preferred_element_type=jnp.float32)
        + mb0_ref[...], 0.0)
    h = jnp.dot(t, mw1_ref[...], preferred_element_type=jnp.float32) + mb1_ref[...]
    hsn_ref[...] = jnp.dot(h, wn_ref[...], preferred_element_type=jnp.float32) * dinv

    @pl.when(i == 0)
    def _():
        sum_ref[...] = jnp.zeros_like(sum_ref)

    sum_ref[...] += jnp.sum(h, axis=0, keepdims=True)


def _tc_b(accparts, hs, dinv, b, mw0, mb0, mw1, mb1, wn):
    return pl.pallas_call(
        _tc_b_body,
        grid=(GRID,),
        in_specs=[
            pl.BlockSpec((NC, BLK, D), lambda i: (0, i, 0)),
            pl.BlockSpec((BLK, D), lambda i: (i, 0)),
            pl.BlockSpec((BLK, 1), lambda i: (i, 0)),
            pl.BlockSpec((1, D), lambda i: (0, 0)),
            pl.BlockSpec((D, 8), lambda i: (0, 0)),
            pl.BlockSpec((1, 8), lambda i: (0, 0)),
            pl.BlockSpec((8, D), lambda i: (0, 0)),
            pl.BlockSpec((1, D), lambda i: (0, 0)),
            pl.BlockSpec((D, D), lambda i: (0, 0)),
        ],
        out_specs=[
            pl.BlockSpec((BLK, D), lambda i: (i, 0)),
            pl.BlockSpec((1, D), lambda i: (0, 0)),
        ],
        out_shape=[
            jax.ShapeDtypeStruct((N, D), jnp.float32),
            jax.ShapeDtypeStruct((1, D), jnp.float32),
        ],
    )(accparts, hs, dinv, b, mw0, mb0, mw1, mb1, wn)


def _tc_c_body(acc_ref, hs_ref, dinv_ref, b_ref, mw0_ref, mb0_ref,
               mw1_ref, mb1_ref, s1_ref, w1t_ref, w2t_ref, bt_ref,
               out_ref, sum_ref):
    i = pl.program_id(0)
    dinv = dinv_ref[...]
    g = dinv * (acc_ref[0] + acc_ref[1] + hs_ref[...]) + b_ref[...]
    t = jnp.maximum(
        jnp.dot(g, mw0_ref[...], preferred_element_type=jnp.float32)
        + mb0_ref[...], 0.0)
    h = jnp.dot(t, mw1_ref[...], preferred_element_type=jnp.float32) + mb1_ref[...]

    @pl.when(i == 0)
    def _():
        sum_ref[...] = jnp.zeros_like(sum_ref)

    sum_ref[...] += jnp.sum(h, axis=0, keepdims=True)

    @pl.when(i == GRID - 1)
    def _():
        logits = (
            jnp.dot(s1_ref[...], w1t_ref[...], preferred_element_type=jnp.float32)
            + jnp.dot(sum_ref[...], w2t_ref[...], preferred_element_type=jnp.float32)
            + bt_ref[...])
        out_ref[...] = jax.nn.sigmoid(logits)


def _tc_c(accparts, hs, dinv, b, mw0, mb0, mw1, mb1, s1, w1t, w2t, bt):
    return pl.pallas_call(
        _tc_c_body,
        grid=(GRID,),
        in_specs=[
            pl.BlockSpec((NC, BLK, D), lambda i: (0, i, 0)),
            pl.BlockSpec((BLK, D), lambda i: (i, 0)),
            pl.BlockSpec((BLK, 1), lambda i: (i, 0)),
            pl.BlockSpec((1, D), lambda i: (0, 0)),
            pl.BlockSpec((D, 8), lambda i: (0, 0)),
            pl.BlockSpec((1, 8), lambda i: (0, 0)),
            pl.BlockSpec((8, D), lambda i: (0, 0)),
            pl.BlockSpec((1, D), lambda i: (0, 0)),
            pl.BlockSpec((1, D), lambda i: (0, 0)),
            pl.BlockSpec((D, 2), lambda i: (0, 0)),
            pl.BlockSpec((D, 2), lambda i: (0, 0)),
            pl.BlockSpec((1, 2), lambda i: (0, 0)),
        ],
        out_specs=[
            pl.BlockSpec((1, 2), lambda i: (0, 0)),
            pl.BlockSpec((1, D), lambda i: (0, 0)),
        ],
        out_shape=[
            jax.ShapeDtypeStruct((1, 2), jnp.float32),
            jax.ShapeDtypeStruct((1, D), jnp.float32),
        ],
    )(accparts, hs, dinv, b, mw0, mb0, mw1, mb1, s1, w1t, w2t, bt)


def kernel(x, edge_index, gcn0_W, gcn0_b, gcn1_W, gcn1_b,
           mlp0_W0, mlp0_b0, mlp0_W1, mlp0_b1,
           mlp1_W0, mlp1_b0, mlp1_W1, mlp1_b1,
           tcl_f0, tcl_f1, tcl_f2, tcl_b, pi_hidden,
           attend_W, attend_b, out_W, out_b):
    f32 = jnp.float32
    src2d = edge_index[0].reshape(EROWS, CH)
    dst2d = edge_index[1].reshape(EROWS, CH)

    ones_deg = jnp.ones((CH, DEG_W), f32)
    zero_deg = jnp.zeros((ACC_N, DEG_W), f32)
    zero_acc = jnp.zeros((ACC_N, D), f32)

    wA = attend_W[:8, 0]
    wB = attend_W[8:, 0]
    g0v = tcl_f0.T @ wA
    Cmat = (jnp.einsum('d,dyz->yz', wA, tcl_b)
            + jnp.einsum('f,fyz->yz', wB, pi_hidden) + attend_b[0])
    Cvec = Cmat.T.reshape(1, 64)
    Kmat = jnp.einsum('yb,zc->bczy', tcl_f1, tcl_f2).reshape(64, 64)
    Wtail = Kmat @ out_W
    bt = Cvec @ out_W + out_b[None, :]
    w1t = (g0v[0] / N) * Wtail
    w2t = (g0v[1] / N) * Wtail

    degparts = _sc_deg(dst2d, ones_deg, zero_deg)
    hs0, dinv = _tc_a(x, gcn0_W, degparts)
    acc0 = _sc_conv(hs0, src2d, dst2d, zero_acc)
    hs1, s1 = _tc_b(acc0, hs0, dinv, gcn0_b[None, :],
                    mlp0_W0, mlp0_b0[None, :], mlp0_W1, mlp0_b1[None, :],
                    gcn1_W)
    acc1 = _sc_conv(hs1, src2d, dst2d, zero_acc)
    out, _ = _tc_c(acc1, hs1, dinv, gcn1_b[None, :],
                   mlp1_W0, mlp1_b0[None, :], mlp1_W1, mlp1_b1[None, :],
                   s1, w1t, w2t, bt)
    return out


# speedup vs baseline: 1.0993x; 1.0339x over previous
import functools
import jax
import jax.numpy as jnp
from jax import lax
from jax.experimental import pallas as pl
from jax.experimental.pallas import tpu as pltpu
from jax.experimental.pallas import tpu_sc as plsc

N = 10000
E = 320000
D = 64
NC = 2
NS = 16
NW = NC * NS
CH = 128
EROWS = E // CH
CHB = EROWS // NW
CHX = EROWS - CHB * NW
IDXR = CHB + 1
ACC_N = 10112
ROWS_PER_SUB = ACC_N // NS
DEG_W = 16
BLK = 2000
GRID = N // BLK

_mesh = plsc.VectorSubcoreMesh(core_axis_name="c", subcore_axis_name="s")


def _sc_deg_body(dst_hbm, ones_hbm, zero_hbm, out_hbm, didx, ones_v, acc, sem):
    c = lax.axis_index("c")
    s = lax.axis_index("s")
    w = s * NC + c
    base = CHB * w + jnp.minimum(w, CHX)
    nch = CHB + (w < CHX).astype(jnp.int32)
    r0 = s * ROWS_PER_SUB
    pltpu.async_copy(zero_hbm.at[pl.ds(r0, ROWS_PER_SUB)],
                     acc.at[pl.ds(r0, ROWS_PER_SUB)], sem)
    pltpu.async_copy(dst_hbm.at[pl.ds(base, CHB)], didx.at[pl.ds(0, CHB)],
                     sem)

    @pl.when(w < CHX)
    def _():
        pltpu.sync_copy(dst_hbm.at[pl.ds(base + CHB, 1)],
                        didx.at[pl.ds(CHB, 1)])

    pltpu.async_copy(ones_hbm, ones_v, sem)
    pltpu.make_async_copy(zero_hbm.at[pl.ds(r0, ROWS_PER_SUB)],
                          acc.at[pl.ds(r0, ROWS_PER_SUB)], sem).wait()
    pltpu.make_async_copy(dst_hbm.at[pl.ds(base, CHB)],
                          didx.at[pl.ds(0, CHB)], sem).wait()
    pltpu.make_async_copy(ones_hbm, ones_v, sem).wait()
    plsc.subcore_barrier()

    def body(j, carry):
        @pl.when(j >= 16)
        def _():
            pltpu.make_async_copy(ones_v, acc.at[didx.at[0]], sem).wait()

        pltpu.async_copy(ones_v, acc.at[didx.at[j]], sem, add=True)
        return carry

    lax.fori_loop(0, nch, body, 0)

    def drain(j, carry):
        pltpu.make_async_copy(ones_v, acc.at[didx.at[0]], sem).wait()
        return carry

    lax.fori_loop(0, 16, drain, 0)
    plsc.subcore_barrier()
    pltpu.sync_copy(acc.at[pl.ds(r0, ROWS_PER_SUB)],
                    out_hbm.at[c, pl.ds(r0, ROWS_PER_SUB)])


_sc_deg = pl.kernel(
    _sc_deg_body,
    out_type=jax.ShapeDtypeStruct((NC, ACC_N, DEG_W), jnp.float32),
    mesh=_mesh,
    compiler_params=pltpu.CompilerParams(use_tc_tiling_on_sc=False),
    scratch_types=[
        pltpu.VMEM((IDXR, CH), jnp.int32),
        pltpu.VMEM((CH, DEG_W), jnp.float32),
        pltpu.VMEM_SHARED((ACC_N, DEG_W), jnp.float32),
        pltpu.SemaphoreType.DMA,
    ],
)


NBUF = 3
LOOKAHEAD = 2


def _sc_conv_body(hs_hbm, src_hbm, dst_hbm, zero_hbm, out_hbm,
                  sidx, didx, rows, hs_sp, acc, gsem, ssem):
    c = lax.axis_index("c")
    s = lax.axis_index("s")
    w = s * NC + c
    base = CHB * w + jnp.minimum(w, CHX)
    nch = CHB + (w < CHX).astype(jnp.int32)
    r0 = s * ROWS_PER_SUB
    pltpu.async_copy(zero_hbm.at[pl.ds(r0, ROWS_PER_SUB)],
                     acc.at[pl.ds(r0, ROWS_PER_SUB)], gsem.at[0])

    @pl.when(s < 10)
    def _():
        pltpu.async_copy(hs_hbm.at[pl.ds(s * 1000, 1000)],
                         hs_sp.at[pl.ds(s * 1000, 1000)], gsem.at[1])

    pltpu.async_copy(src_hbm.at[pl.ds(base, CHB)], sidx.at[pl.ds(0, CHB)],
                     ssem.at[0])
    pltpu.async_copy(dst_hbm.at[pl.ds(base, CHB)], didx.at[pl.ds(0, CHB)],
                     ssem.at[1])

    @pl.when(w < CHX)
    def _():
        pltpu.sync_copy(src_hbm.at[pl.ds(base + CHB, 1)],
                        sidx.at[pl.ds(CHB, 1)])
        pltpu.sync_copy(dst_hbm.at[pl.ds(base + CHB, 1)],
                        didx.at[pl.ds(CHB, 1)])

    pltpu.make_async_copy(zero_hbm.at[pl.ds(r0, ROWS_PER_SUB)],
                          acc.at[pl.ds(r0, ROWS_PER_SUB)], gsem.at[0]).wait()

    @pl.when(s < 10)
    def _():
        pltpu.make_async_copy(hs_hbm.at[pl.ds(s * 1000, 1000)],
                              hs_sp.at[pl.ds(s * 1000, 1000)],
                              gsem.at[1]).wait()

    pltpu.make_async_copy(src_hbm.at[pl.ds(base, CHB)],
                          sidx.at[pl.ds(0, CHB)], ssem.at[0]).wait()
    pltpu.make_async_copy(dst_hbm.at[pl.ds(base, CHB)],
                          didx.at[pl.ds(0, CHB)], ssem.at[1]).wait()
    plsc.subcore_barrier()

    def body(j, carry):
        b = lax.rem(j, NBUF)

        @pl.when(jnp.logical_and(j >= NBUF, j < nch))
        def _():
            pltpu.make_async_copy(rows.at[b], acc.at[didx.at[j]],
                                  ssem.at[b]).wait()

        @pl.when(j < nch)
        def _():
            pltpu.async_copy(hs_sp.at[sidx.at[j]], rows.at[b], gsem.at[b])

        jk = j - LOOKAHEAD

        @pl.when(jk >= 0)
        def _():
            bk = lax.rem(jk, NBUF)
            pltpu.make_async_copy(hs_sp.at[sidx.at[jk]], rows.at[bk],
                                  gsem.at[bk]).wait()
            pltpu.async_copy(rows.at[bk], acc.at[didx.at[jk]], ssem.at[bk],
                             add=True)

        return carry

    lax.fori_loop(0, nch + LOOKAHEAD, body, 0)

    def drain(b, carry):
        pltpu.make_async_copy(rows.at[b], acc.at[didx.at[0]],
                              ssem.at[b]).wait()
        return carry

    lax.fori_loop(0, NBUF, drain, 0)
    plsc.subcore_barrier()
    pltpu.sync_copy(acc.at[pl.ds(r0, ROWS_PER_SUB)],
                    out_hbm.at[c, pl.ds(r0, ROWS_PER_SUB)])


_sc_conv = pl.kernel(
    _sc_conv_body,
    out_type=jax.ShapeDtypeStruct((NC, ACC_N, D), jnp.float32),
    mesh=_mesh,
    compiler_params=pltpu.CompilerParams(use_tc_tiling_on_sc=False),
    scratch_types=[
        pltpu.VMEM((IDXR, CH), jnp.int32),
        pltpu.VMEM((IDXR, CH), jnp.int32),
        pltpu.VMEM((NBUF, CH, D), jnp.float32),
        pltpu.VMEM_SHARED((N, D), jnp.float32),
        pltpu.VMEM_SHARED((ACC_N, D), jnp.float32),
        pltpu.SemaphoreType.DMA((NBUF,)),
        pltpu.SemaphoreType.DMA((NBUF,)),
    ],
)


def _tc_a_body(x_ref, w0_ref, deg_ref, hs0_ref, dinv_ref):
    deg = deg_ref[0, :, 0:1] + deg_ref[1, :, 0:1] + 1.0
    dinv = lax.rsqrt(deg)
    h0 = jnp.dot(x_ref[...], w0_ref[...], preferred_element_type=jnp.float32)
    hs0_ref[...] = h0 * dinv
    dinv_ref[...] = dinv


def _tc_a(x, w0, degparts):
    return pl.pallas_call(
        _tc_a_body,
        grid=(GRID,),
        in_specs=[
            pl.BlockSpec((BLK, 128), lambda i: (i, 0)),
            pl.BlockSpec((128, D), lambda i: (0, 0)),
            pl.BlockSpec((NC, BLK, DEG_W), lambda i: (0, i, 0)),
        ],
        out_specs=[
            pl.BlockSpec((BLK, D), lambda i: (i, 0)),
            pl.BlockSpec((BLK, 1), lambda i: (i, 0)),
        ],
        out_shape=[
            jax.ShapeDtypeStruct((N, D), jnp.float32),
            jax.ShapeDtypeStruct((N, 1), jnp.float32),
        ],
    )(x, w0, degparts)


def _tc_b_body(acc_ref, hs_ref, dinv_ref, b_ref, mw0_ref, mb0_ref,
               mw1_ref, mb1_ref, wn_ref, hsn_ref, sum_ref):
    i = pl.program_id(0)
    dinv = dinv_ref[...]
    g = dinv * (acc_ref[0] + acc_ref[1] + hs_ref[...]) + b_ref[...]
    t = jnp.maximum(
        jnp.dot(g, mw0_ref[...], preferred_element_type=jnp.float32)
        + mb0_ref[...], 0.0)
    h = jnp.dot(t, mw1_ref[...], preferred_element_type=jnp.float32) + mb1_ref[...]
    hsn_ref[...] = jnp.dot(h, wn_ref[...], preferred_element_type=jnp.float32) * dinv

    @pl.when(i == 0)
    def _():
        sum_ref[...] = jnp.zeros_like(sum_ref)

    sum_ref[...] += jnp.sum(h, axis=0, keepdims=True)


def _tc_b(accparts, hs, dinv, b, mw0, mb0, mw1, mb1, wn):
    return pl.pallas_call(
        _tc_b_body,
        grid=(GRID,),
        in_specs=[
            pl.BlockSpec((NC, BLK, D), lambda i: (0, i, 0)),
            pl.BlockSpec((BLK, D), lambda i: (i, 0)),
            pl.BlockSpec((BLK, 1), lambda i: (i, 0)),
            pl.BlockSpec((1, D), lambda i: (0, 0)),
            pl.BlockSpec((D, 8), lambda i: (0, 0)),
            pl.BlockSpec((1, 8), lambda i: (0, 0)),
            pl.BlockSpec((8, D), lambda i: (0, 0)),
            pl.BlockSpec((1, D), lambda i: (0, 0)),
            pl.BlockSpec((D, D), lambda i: (0, 0)),
        ],
        out_specs=[
            pl.BlockSpec((BLK, D), lambda i: (i, 0)),
            pl.BlockSpec((1, D), lambda i: (0, 0)),
        ],
        out_shape=[
            jax.ShapeDtypeStruct((N, D), jnp.float32),
            jax.ShapeDtypeStruct((1, D), jnp.float32),
        ],
    )(accparts, hs, dinv, b, mw0, mb0, mw1, mb1, wn)


def _tc_c_body(acc_ref, hs_ref, dinv_ref, b_ref, mw0_ref, mb0_ref,
               mw1_ref, mb1_ref, s1_ref, w1t_ref, w2t_ref, bt_ref,
               out_ref, sum_ref):
    i = pl.program_id(0)
    dinv = dinv_ref[...]
    g = dinv * (acc_ref[0] + acc_ref[1] + hs_ref[...]) + b_ref[...]
    t = jnp.maximum(
        jnp.dot(g, mw0_ref[...], preferred_element_type=jnp.float32)
        + mb0_ref[...], 0.0)
    h = jnp.dot(t, mw1_ref[...], preferred_element_type=jnp.float32) + mb1_ref[...]

    @pl.when(i == 0)
    def _():
        sum_ref[...] = jnp.zeros_like(sum_ref)

    sum_ref[...] += jnp.sum(h, axis=0, keepdims=True)

    @pl.when(i == GRID - 1)
    def _():
        logits = (
            jnp.dot(s1_ref[...], w1t_ref[...], preferred_element_type=jnp.float32)
            + jnp.dot(sum_ref[...], w2t_ref[...], preferred_element_type=jnp.float32)
            + bt_ref[...])
        out_ref[...] = jax.nn.sigmoid(logits)


def _tc_c(accparts, hs, dinv, b, mw0, mb0, mw1, mb1, s1, w1t, w2t, bt):
    return pl.pallas_call(
        _tc_c_body,
        grid=(GRID,),
        in_specs=[
            pl.BlockSpec((NC, BLK, D), lambda i: (0, i, 0)),
            pl.BlockSpec((BLK, D), lambda i: (i, 0)),
            pl.BlockSpec((BLK, 1), lambda i: (i, 0)),
            pl.BlockSpec((1, D), lambda i: (0, 0)),
            pl.BlockSpec((D, 8), lambda i: (0, 0)),
            pl.BlockSpec((1, 8), lambda i: (0, 0)),
            pl.BlockSpec((8, D), lambda i: (0, 0)),
            pl.BlockSpec((1, D), lambda i: (0, 0)),
            pl.BlockSpec((1, D), lambda i: (0, 0)),
            pl.BlockSpec((D, 2), lambda i: (0, 0)),
            pl.BlockSpec((D, 2), lambda i: (0, 0)),
            pl.BlockSpec((1, 2), lambda i: (0, 0)),
        ],
        out_specs=[
            pl.BlockSpec((1, 2), lambda i: (0, 0)),
            pl.BlockSpec((1, D), lambda i: (0, 0)),
        ],
        out_shape=[
            jax.ShapeDtypeStruct((1, 2), jnp.float32),
            jax.ShapeDtypeStruct((1, D), jnp.float32),
        ],
    )(accparts, hs, dinv, b, mw0, mb0, mw1, mb1, s1, w1t, w2t, bt)


def kernel(x, edge_index, gcn0_W, gcn0_b, gcn1_W, gcn1_b,
           mlp0_W0, mlp0_b0, mlp0_W1, mlp0_b1,
           mlp1_W0, mlp1_b0, mlp1_W1, mlp1_b1,
           tcl_f0, tcl_f1, tcl_f2, tcl_b, pi_hidden,
           attend_W, attend_b, out_W, out_b):
    f32 = jnp.float32
    src2d = edge_index[0].reshape(EROWS, CH)
    dst2d = edge_index[1].reshape(EROWS, CH)

    ones_deg = jnp.ones((CH, DEG_W), f32)
    zero_deg = jnp.zeros((ACC_N, DEG_W), f32)
    zero_acc = jnp.zeros((ACC_N, D), f32)

    wA = attend_W[:8, 0]
    wB = attend_W[8:, 0]
    g0v = tcl_f0.T @ wA
    Cmat = (jnp.einsum('d,dyz->yz', wA, tcl_b)
            + jnp.einsum('f,fyz->yz', wB, pi_hidden) + attend_b[0])
    Cvec = Cmat.T.reshape(1, 64)
    Kmat = jnp.einsum('yb,zc->bczy', tcl_f1, tcl_f2).reshape(64, 64)
    Wtail = Kmat @ out_W
    bt = Cvec @ out_W + out_b[None, :]
    w1t = (g0v[0] / N) * Wtail
    w2t = (g0v[1] / N) * Wtail

    degparts = _sc_deg(dst2d, ones_deg, zero_deg)
    hs0, dinv = _tc_a(x, gcn0_W, degparts)
    acc0 = _sc_conv(hs0, src2d, dst2d, zero_acc)
    hs1, s1 = _tc_b(acc0, hs0, dinv, gcn0_b[None, :],
                    mlp0_W0, mlp0_b0[None, :], mlp0_W1, mlp0_b1[None, :],
                    gcn1_W)
    acc1 = _sc_conv(hs1, src2d, dst2d, zero_acc)
    out, _ = _tc_c(acc1, hs1, dinv, gcn1_b[None, :],
                   mlp1_W0, mlp1_b0[None, :], mlp1_W1, mlp1_b1[None, :],
                   s1, w1t, w2t, bt)
    return out


# speedup vs baseline: 1.1514x; 1.0474x over previous
import functools
import jax
import jax.numpy as jnp
from jax import lax
from jax.experimental import pallas as pl
from jax.experimental.pallas import tpu as pltpu
from jax.experimental.pallas import tpu_sc as plsc

N = 10000
E = 320000
D = 64
NC = 2
NS = 16
NW = NC * NS
CH = 128
EROWS = E // CH
CHB = EROWS // NW
CHX = EROWS - CHB * NW
IDXR = CHB + 1
ACC_N = 10112
ROWS_PER_SUB = ACC_N // NS
DEG_W = 16
BLK = 2000
GRID = N // BLK

_mesh = plsc.VectorSubcoreMesh(core_axis_name="c", subcore_axis_name="s")


def _sc_deg_body(e_hbm, ones_hbm, zero_hbm, out_hbm, didx, ones_v, acc, sem):
    c = lax.axis_index("c")
    s = lax.axis_index("s")
    w = s * NC + c
    base = CHB * w + jnp.minimum(w, CHX)
    nch = CHB + (w < CHX).astype(jnp.int32)
    r0 = s * ROWS_PER_SUB
    pltpu.async_copy(zero_hbm.at[pl.ds(r0, ROWS_PER_SUB)],
                     acc.at[pl.ds(r0, ROWS_PER_SUB)], sem)
    pltpu.async_copy(e_hbm.at[1, pl.ds(base, CHB)], didx.at[pl.ds(0, CHB)],
                     sem)

    @pl.when(w < CHX)
    def _():
        pltpu.sync_copy(e_hbm.at[1, pl.ds(base + CHB, 1)],
                        didx.at[pl.ds(CHB, 1)])

    pltpu.async_copy(ones_hbm, ones_v, sem)
    pltpu.make_async_copy(zero_hbm.at[pl.ds(r0, ROWS_PER_SUB)],
                          acc.at[pl.ds(r0, ROWS_PER_SUB)], sem).wait()
    pltpu.make_async_copy(e_hbm.at[1, pl.ds(base, CHB)],
                          didx.at[pl.ds(0, CHB)], sem).wait()
    pltpu.make_async_copy(ones_hbm, ones_v, sem).wait()
    plsc.subcore_barrier()

    def body(j, carry):
        @pl.when(j >= 16)
        def _():
            pltpu.make_async_copy(ones_v, acc.at[didx.at[0]], sem).wait()

        pltpu.async_copy(ones_v, acc.at[didx.at[j]], sem, add=True)
        return carry

    lax.fori_loop(0, nch, body, 0)

    def drain(j, carry):
        pltpu.make_async_copy(ones_v, acc.at[didx.at[0]], sem).wait()
        return carry

    lax.fori_loop(0, 16, drain, 0)
    plsc.subcore_barrier()
    pltpu.sync_copy(acc.at[pl.ds(r0, ROWS_PER_SUB)],
                    out_hbm.at[c, pl.ds(r0, ROWS_PER_SUB)])


_sc_deg = pl.kernel(
    _sc_deg_body,
    out_type=jax.ShapeDtypeStruct((NC, ACC_N, DEG_W), jnp.float32),
    mesh=_mesh,
    compiler_params=pltpu.CompilerParams(use_tc_tiling_on_sc=False),
    scratch_types=[
        pltpu.VMEM((IDXR, CH), jnp.int32),
        pltpu.VMEM((CH, DEG_W), jnp.float32),
        pltpu.VMEM_SHARED((ACC_N, DEG_W), jnp.float32),
        pltpu.SemaphoreType.DMA,
    ],
)


NBUF = 3
LOOKAHEAD = 2


def _sc_conv_body(hs_hbm, e_hbm, zero_hbm, out_hbm,
                  sidx, didx, rows, hs_sp, acc, gsem, ssem):
    c = lax.axis_index("c")
    s = lax.axis_index("s")
    w = s * NC + c
    base = CHB * w + jnp.minimum(w, CHX)
    nch = CHB + (w < CHX).astype(jnp.int32)
    r0 = s * ROWS_PER_SUB
    pltpu.async_copy(zero_hbm.at[pl.ds(r0, ROWS_PER_SUB)],
                     acc.at[pl.ds(r0, ROWS_PER_SUB)], gsem.at[0])

    @pl.when(s < 10)
    def _():
        pltpu.async_copy(hs_hbm.at[pl.ds(s * 1000, 1000)],
                         hs_sp.at[pl.ds(s * 1000, 1000)], gsem.at[1])

    pltpu.async_copy(e_hbm.at[0, pl.ds(base, CHB)], sidx.at[pl.ds(0, CHB)],
                     ssem.at[0])
    pltpu.async_copy(e_hbm.at[1, pl.ds(base, CHB)], didx.at[pl.ds(0, CHB)],
                     ssem.at[1])

    @pl.when(w < CHX)
    def _():
        pltpu.sync_copy(e_hbm.at[0, pl.ds(base + CHB, 1)],
                        sidx.at[pl.ds(CHB, 1)])
        pltpu.sync_copy(e_hbm.at[1, pl.ds(base + CHB, 1)],
                        didx.at[pl.ds(CHB, 1)])

    pltpu.make_async_copy(zero_hbm.at[pl.ds(r0, ROWS_PER_SUB)],
                          acc.at[pl.ds(r0, ROWS_PER_SUB)], gsem.at[0]).wait()

    @pl.when(s < 10)
    def _():
        pltpu.make_async_copy(hs_hbm.at[pl.ds(s * 1000, 1000)],
                              hs_sp.at[pl.ds(s * 1000, 1000)],
                              gsem.at[1]).wait()

    pltpu.make_async_copy(e_hbm.at[0, pl.ds(base, CHB)],
                          sidx.at[pl.ds(0, CHB)], ssem.at[0]).wait()
    pltpu.make_async_copy(e_hbm.at[1, pl.ds(base, CHB)],
                          didx.at[pl.ds(0, CHB)], ssem.at[1]).wait()
    plsc.subcore_barrier()

    def body(j, carry):
        b = lax.rem(j, NBUF)

        @pl.when(jnp.logical_and(j >= NBUF, j < nch))
        def _():
            pltpu.make_async_copy(rows.at[b], acc.at[didx.at[j]],
                                  ssem.at[b]).wait()

        @pl.when(j < nch)
        def _():
            pltpu.async_copy(hs_sp.at[sidx.at[j]], rows.at[b], gsem.at[b])

        jk = j - LOOKAHEAD

        @pl.when(jk >= 0)
        def _():
            bk = lax.rem(jk, NBUF)
            pltpu.make_async_copy(hs_sp.at[sidx.at[jk]], rows.at[bk],
                                  gsem.at[bk]).wait()
            pltpu.async_copy(rows.at[bk], acc.at[didx.at[jk]], ssem.at[bk],
                             add=True)

        return carry

    lax.fori_loop(0, nch + LOOKAHEAD, body, 0)

    def drain(b, carry):
        pltpu.make_async_copy(rows.at[b], acc.at[didx.at[0]],
                              ssem.at[b]).wait()
        return carry

    lax.fori_loop(0, NBUF, drain, 0)
    plsc.subcore_barrier()
    pltpu.sync_copy(acc.at[pl.ds(r0, ROWS_PER_SUB)],
                    out_hbm.at[c, pl.ds(r0, ROWS_PER_SUB)])


_sc_conv = pl.kernel(
    _sc_conv_body,
    out_type=jax.ShapeDtypeStruct((NC, ACC_N, D), jnp.float32),
    mesh=_mesh,
    compiler_params=pltpu.CompilerParams(use_tc_tiling_on_sc=False),
    scratch_types=[
        pltpu.VMEM((IDXR, CH), jnp.int32),
        pltpu.VMEM((IDXR, CH), jnp.int32),
        pltpu.VMEM((NBUF, CH, D), jnp.float32),
        pltpu.VMEM_SHARED((N, D), jnp.float32),
        pltpu.VMEM_SHARED((ACC_N, D), jnp.float32),
        pltpu.SemaphoreType.DMA((NBUF,)),
        pltpu.SemaphoreType.DMA((NBUF,)),
    ],
)


def _tc_a_body(x_ref, w0_ref, deg_ref, hs0_ref, dinv_ref):
    deg = deg_ref[0, :, 0:1] + deg_ref[1, :, 0:1] + 1.0
    dinv = lax.rsqrt(deg)
    h0 = jnp.dot(x_ref[...], w0_ref[...], preferred_element_type=jnp.float32)
    hs0_ref[...] = h0 * dinv
    dinv_ref[...] = dinv


def _tc_a(x, w0, degparts):
    return pl.pallas_call(
        _tc_a_body,
        grid=(GRID,),
        in_specs=[
            pl.BlockSpec((BLK, 128), lambda i: (i, 0)),
            pl.BlockSpec((128, D), lambda i: (0, 0)),
            pl.BlockSpec((NC, BLK, DEG_W), lambda i: (0, i, 0)),
        ],
        out_specs=[
            pl.BlockSpec((BLK, D), lambda i: (i, 0)),
            pl.BlockSpec((BLK, 1), lambda i: (i, 0)),
        ],
        out_shape=[
            jax.ShapeDtypeStruct((N, D), jnp.float32),
            jax.ShapeDtypeStruct((N, 1), jnp.float32),
        ],
    )(x, w0, degparts)


def _tc_b_body(acc_ref, hs_ref, dinv_ref, b_ref, mw0_ref, mb0_ref,
               mw1_ref, mb1_ref, wn_ref, hsn_ref, sum_ref):
    i = pl.program_id(0)
    dinv = dinv_ref[...]
    g = dinv * (acc_ref[0] + acc_ref[1] + hs_ref[...]) + b_ref[...]
    t = jnp.maximum(
        jnp.dot(g, mw0_ref[...], preferred_element_type=jnp.float32)
        + mb0_ref[...], 0.0)
    h = jnp.dot(t, mw1_ref[...], preferred_element_type=jnp.float32) + mb1_ref[...]
    hsn_ref[...] = jnp.dot(h, wn_ref[...], preferred_element_type=jnp.float32) * dinv

    @pl.when(i == 0)
    def _():
        sum_ref[...] = jnp.zeros_like(sum_ref)

    sum_ref[...] += jnp.sum(h, axis=0, keepdims=True)


def _tc_b(accparts, hs, dinv, b, mw0, mb0, mw1, mb1, wn):
    return pl.pallas_call(
        _tc_b_body,
        grid=(GRID,),
        in_specs=[
            pl.BlockSpec((NC, BLK, D), lambda i: (0, i, 0)),
            pl.BlockSpec((BLK, D), lambda i: (i, 0)),
            pl.BlockSpec((BLK, 1), lambda i: (i, 0)),
            pl.BlockSpec((1, D), lambda i: (0, 0)),
            pl.BlockSpec((D, 8), lambda i: (0, 0)),
            pl.BlockSpec((1, 8), lambda i: (0, 0)),
            pl.BlockSpec((8, D), lambda i: (0, 0)),
            pl.BlockSpec((1, D), lambda i: (0, 0)),
            pl.BlockSpec((D, D), lambda i: (0, 0)),
        ],
        out_specs=[
            pl.BlockSpec((BLK, D), lambda i: (i, 0)),
            pl.BlockSpec((1, D), lambda i: (0, 0)),
        ],
        out_shape=[
            jax.ShapeDtypeStruct((N, D), jnp.float32),
            jax.ShapeDtypeStruct((1, D), jnp.float32),
        ],
    )(accparts, hs, dinv, b, mw0, mb0, mw1, mb1, wn)


def _tc_c_body(acc_ref, hs_ref, dinv_ref, b_ref, mw0_ref, mb0_ref,
               mw1_ref, mb1_ref, s1_ref, w1t_ref, w2t_ref, bt_ref,
               out_ref, sum_ref):
    i = pl.program_id(0)
    dinv = dinv_ref[...]
    g = dinv * (acc_ref[0] + acc_ref[1] + hs_ref[...]) + b_ref[...]
    t = jnp.maximum(
        jnp.dot(g, mw0_ref[...], preferred_element_type=jnp.float32)
        + mb0_ref[...], 0.0)
    h = jnp.dot(t, mw1_ref[...], preferred_element_type=jnp.float32) + mb1_ref[...]

    @pl.when(i == 0)
    def _():
        sum_ref[...] = jnp.zeros_like(sum_ref)

    sum_ref[...] += jnp.sum(h, axis=0, keepdims=True)

    @pl.when(i == GRID - 1)
    def _():
        logits = (
            jnp.dot(s1_ref[...], w1t_ref[...], preferred_element_type=jnp.float32)
            + jnp.dot(sum_ref[...], w2t_ref[...], preferred_element_type=jnp.float32)
            + bt_ref[...])
        out_ref[...] = jax.nn.sigmoid(logits)


def _tc_c(accparts, hs, dinv, b, mw0, mb0, mw1, mb1, s1, w1t, w2t, bt):
    return pl.pallas_call(
        _tc_c_body,
        grid=(GRID,),
        in_specs=[
            pl.BlockSpec((NC, BLK, D), lambda i: (0, i, 0)),
            pl.BlockSpec((BLK, D), lambda i: (i, 0)),
            pl.BlockSpec((BLK, 1), lambda i: (i, 0)),
            pl.BlockSpec((1, D), lambda i: (0, 0)),
            pl.BlockSpec((D, 8), lambda i: (0, 0)),
            pl.BlockSpec((1, 8), lambda i: (0, 0)),
            pl.BlockSpec((8, D), lambda i: (0, 0)),
            pl.BlockSpec((1, D), lambda i: (0, 0)),
            pl.BlockSpec((1, D), lambda i: (0, 0)),
            pl.BlockSpec((D, 2), lambda i: (0, 0)),
            pl.BlockSpec((D, 2), lambda i: (0, 0)),
            pl.BlockSpec((1, 2), lambda i: (0, 0)),
        ],
        out_specs=[
            pl.BlockSpec((1, 2), lambda i: (0, 0)),
            pl.BlockSpec((1, D), lambda i: (0, 0)),
        ],
        out_shape=[
            jax.ShapeDtypeStruct((1, 2), jnp.float32),
            jax.ShapeDtypeStruct((1, D), jnp.float32),
        ],
    )(accparts, hs, dinv, b, mw0, mb0, mw1, mb1, s1, w1t, w2t, bt)


def kernel(x, edge_index, gcn0_W, gcn0_b, gcn1_W, gcn1_b,
           mlp0_W0, mlp0_b0, mlp0_W1, mlp0_b1,
           mlp1_W0, mlp1_b0, mlp1_W1, mlp1_b1,
           tcl_f0, tcl_f1, tcl_f2, tcl_b, pi_hidden,
           attend_W, attend_b, out_W, out_b):
    f32 = jnp.float32
    e3 = edge_index.reshape(2, EROWS, CH)

    ones_deg = jnp.ones((CH, DEG_W), f32)
    zero_deg = jnp.zeros((ACC_N, DEG_W), f32)
    zero_acc = jnp.zeros((ACC_N, D), f32)

    wA = attend_W[:8, 0]
    wB = attend_W[8:, 0]
    g0v = tcl_f0.T @ wA
    Cmat = (jnp.einsum('d,dyz->yz', wA, tcl_b)
            + jnp.einsum('f,fyz->yz', wB, pi_hidden) + attend_b[0])
    Cvec = Cmat.T.reshape(1, 64)
    Kmat = jnp.einsum('yb,zc->bczy', tcl_f1, tcl_f2).reshape(64, 64)
    Wtail = Kmat @ out_W
    bt = Cvec @ out_W + out_b[None, :]
    w1t = (g0v[0] / N) * Wtail
    w2t = (g0v[1] / N) * Wtail

    degparts = _sc_deg(e3, ones_deg, zero_deg)
    hs0, dinv = _tc_a(x, gcn0_W, degparts)
    acc0 = _sc_conv(hs0, e3, zero_acc)
    hs1, s1 = _tc_b(acc0, hs0, dinv, gcn0_b[None, :],
                    mlp0_W0, mlp0_b0[None, :], mlp0_W1, mlp0_b1[None, :],
                    gcn1_W)
    acc1 = _sc_conv(hs1, e3, zero_acc)
    out, _ = _tc_c(acc1, hs1, dinv, gcn1_b[None, :],
                   mlp1_W0, mlp1_b0[None, :], mlp1_W1, mlp1_b1[None, :],
                   s1, w1t, w2t, bt)
    return out
